# baseline (speedup 1.0000x reference)
import functools

import jax
import jax.numpy as jnp
from jax import lax
from jax.experimental import pallas as pl
from jax.experimental.pallas import tpu as pltpu
from jax.experimental.pallas import tpu_sc as plsc

VOCAB = 49408
NPOS = 77
D = 768
ROWS_TOTAL = 4096 * 77
NC, NS, L = 2, 16, 16
NWORK = NC * NS
ROWS_PER_W = ROWS_TOTAL // NWORK
K = 16
NCH = ROWS_PER_W // K
NSLOT = 4

_mesh = plsc.VectorSubcoreMesh(core_axis_name="c", subcore_axis_name="s")


@functools.partial(
    pl.kernel,
    mesh=_mesh,
    out_type=jax.ShapeDtypeStruct((ROWS_TOTAL, D), jnp.float32),
    scratch_types=[
        pltpu.VMEM((NPOS * D,), jnp.float32),
        pltpu.VMEM((ROWS_PER_W,), jnp.int32),
        pltpu.VMEM((ROWS_PER_W,), jnp.int32),
        pltpu.VMEM((NSLOT, K, D), jnp.float32),
        pltpu.SemaphoreType.DMA,
        pltpu.SemaphoreType.DMA,
        pltpu.SemaphoreType.DMA,
        pltpu.SemaphoreType.DMA,
        pltpu.SemaphoreType.DMA,
        pltpu.SemaphoreType.DMA,
        pltpu.SemaphoreType.DMA,
        pltpu.SemaphoreType.DMA,
    ],
    compiler_params=pltpu.CompilerParams(needs_layout_passes=False),
)
def _embed_kernel(tok_hbm, posf_hbm, tid_hbm, pid_hbm, out_hbm,
                  pos_v, tidx_v, pidx_v, buf_v,
                  sg0, sg1, sg2, sg3, so0, so1, so2, so3):
    wid = lax.axis_index("s") * NC + lax.axis_index("c")
    base_w = wid * ROWS_PER_W
    sgs = (sg0, sg1, sg2, sg3)
    sos = (so0, so1, so2, so3)

    pltpu.sync_copy(posf_hbm, pos_v)
    pltpu.sync_copy(tid_hbm.at[pl.ds(base_w, ROWS_PER_W)], tidx_v)
    pltpu.sync_copy(pid_hbm.at[pl.ds(base_w, ROWS_PER_W)], pidx_v)

    def start_gather(g, b):
        pltpu.async_copy(tok_hbm.at[tidx_v.at[pl.ds(g * K, K)]],
                         buf_v.at[b], sgs[b])

    def wait_gather(b):
        pltpu.make_async_copy(tok_hbm.at[pl.ds(0, K)], buf_v.at[b],
                              sgs[b]).wait()

    def start_out(g, b):
        pltpu.async_copy(buf_v.at[b], out_hbm.at[pl.ds(base_w + g * K, K)],
                         sos[b])

    def wait_out(b):
        pltpu.make_async_copy(buf_v.at[b], out_hbm.at[pl.ds(0, K)],
                              sos[b]).wait()

    start_gather(0, 0)
    start_gather(1, 1)

    iota = lax.iota(jnp.int32, L)

    def do_add(g, b):
        def row(r, c):
            rv = jnp.full((L,), g * K + r, jnp.int32)
            pidv = plsc.load_gather(pidx_v, [rv])
            bvec = pidv * D + iota
            for j in range(D // L):
                sl = pl.ds(j * L, L)
                v = plsc.load_gather(pos_v, [bvec + j * L])
                buf_v[b, r, sl] = buf_v[b, r, sl] + v
            return c
        lax.fori_loop(0, K, row, 0)

    def outer(gg, c):
        for b in range(NSLOT):
            g = gg * NSLOT + b
            wait_gather(b)
            do_add(g, b)
            start_out(g, b)

            @pl.when(g >= 2)
            def _():
                wait_out((b - 2) % NSLOT)

            @pl.when(g + 2 < NCH)
            def _():
                start_gather(g + 2, (b + 2) % NSLOT)
        return c

    lax.fori_loop(0, NCH // NSLOT, outer, 0)
    wait_out((NCH - 2) % NSLOT)
    wait_out((NCH - 1) % NSLOT)


def kernel(input_ids, position_ids, token_embedding, position_embedding):
    tid = input_ids.reshape(-1).astype(jnp.int32)
    pid = position_ids.reshape(-1).astype(jnp.int32)
    out = _embed_kernel(token_embedding, position_embedding.reshape(-1),
                        tid, pid)
    return out.reshape(input_ids.shape + (D,))


# speedup vs baseline: 1.0902x; 1.0902x over previous
import functools

import jax
import jax.numpy as jnp
from jax import lax
from jax.experimental import pallas as pl
from jax.experimental.pallas import tpu as pltpu
from jax.experimental.pallas import tpu_sc as plsc

VOCAB = 49408
NPOS = 77
D = 768
ROWS_TOTAL = 4096 * 77
NC, NS, L = 2, 16, 16
NWORK = NC * NS
ROWS_PER_W = ROWS_TOTAL // NWORK
K = 16
NCH = ROWS_PER_W // K
NSLOT = 4

_mesh = plsc.VectorSubcoreMesh(core_axis_name="c", subcore_axis_name="s")


@functools.partial(
    pl.kernel,
    mesh=_mesh,
    out_type=jax.ShapeDtypeStruct((ROWS_TOTAL, D), jnp.float32),
    scratch_types=[
        pltpu.VMEM((NPOS * D,), jnp.float32),
        pltpu.VMEM((ROWS_PER_W,), jnp.int32),
        pltpu.VMEM((ROWS_PER_W,), jnp.int32),
        pltpu.VMEM((NSLOT, K, D), jnp.float32),
        [pltpu.SemaphoreType.DMA] * NSLOT,
        [pltpu.SemaphoreType.DMA] * NSLOT,
    ],
    compiler_params=pltpu.CompilerParams(needs_layout_passes=False),
)
def _embed_kernel(tok_hbm, posf_hbm, tid_hbm, pid_hbm, out_hbm,
                  pos_v, tidx_v, pidx_v, buf_v, sgs, sos):
    wid = lax.axis_index("s") * NC + lax.axis_index("c")
    base_w = wid * ROWS_PER_W

    pltpu.sync_copy(posf_hbm, pos_v)
    pltpu.sync_copy(tid_hbm.at[pl.ds(base_w, ROWS_PER_W)], tidx_v)
    pltpu.sync_copy(pid_hbm.at[pl.ds(base_w, ROWS_PER_W)], pidx_v)

    def start_gather(g, b):
        pltpu.async_copy(tok_hbm.at[tidx_v.at[pl.ds(g * K, K)]],
                         buf_v.at[b], sgs[b])

    def wait_gather(b):
        pltpu.make_async_copy(tok_hbm.at[pl.ds(0, K)], buf_v.at[b],
                              sgs[b]).wait()

    def start_out(g, b):
        pltpu.async_copy(buf_v.at[b], out_hbm.at[pl.ds(base_w + g * K, K)],
                         sos[b])

    def wait_out(b):
        pltpu.make_async_copy(buf_v.at[b], out_hbm.at[pl.ds(0, K)],
                              sos[b]).wait()

    start_gather(0, 0)
    start_gather(1, 1)

    iota = lax.iota(jnp.int32, L)

    def do_add(g, b):
        def row(r, c):
            rv = jnp.full((L,), g * K + r, jnp.int32)
            pidv = plsc.load_gather(pidx_v, [rv])
            bvec = pidv * D + iota
            for j in range(D // L):
                v = plsc.load_gather(pos_v, [bvec + j * L])
                plsc.addupdate(buf_v.at[b, r, pl.ds(j * L, L)], v)
            return c
        lax.fori_loop(0, K, row, 0)

    def outer(gg, c):
        for b in range(NSLOT):
            g = gg * NSLOT + b
            wait_gather(b)
            do_add(g, b)
            start_out(g, b)

            @pl.when(g >= 2)
            def _():
                wait_out((b - 2) % NSLOT)

            @pl.when(g + 2 < NCH)
            def _():
                start_gather(g + 2, (b + 2) % NSLOT)
        return c

    lax.fori_loop(0, NCH // NSLOT, outer, 0)
    wait_out((NCH - 2) % NSLOT)
    wait_out((NCH - 1) % NSLOT)


def kernel(input_ids, position_ids, token_embedding, position_embedding):
    tid = input_ids.reshape(-1).astype(jnp.int32)
    pid = position_ids.reshape(-1).astype(jnp.int32)
    out = _embed_kernel(token_embedding, position_embedding.reshape(-1),
                        tid, pid)
    return out.reshape(input_ids.shape + (D,))


# speedup vs baseline: 1.4400x; 1.3208x over previous
import functools

import jax
import jax.numpy as jnp
from jax import lax
from jax.experimental import pallas as pl
from jax.experimental.pallas import tpu as pltpu
from jax.experimental.pallas import tpu_sc as plsc

VOCAB = 49408
NPOS = 77
NPAD = 80
D = 768
B = 4096
NC, NS, L = 2, 16, 16
NWORK = NC * NS
BB_PER_W = B // NWORK
NSUB = 5
W0 = (0, 16, 32, 48, 64)
TK = 13
NCHT = BB_PER_W * NSUB

_mesh = plsc.VectorSubcoreMesh(core_axis_name="c", subcore_axis_name="s")


@functools.partial(
    pl.kernel,
    mesh=_mesh,
    out_type=jax.ShapeDtypeStruct((B, NPOS, D), jnp.float32),
    scratch_types=[
        pltpu.VMEM((NPOS * D,), jnp.float32),
        pltpu.VMEM((NSUB * L,), jnp.int32),
        pltpu.VMEM((NSUB * L,), jnp.int32),
        pltpu.VMEM((4, L, D), jnp.float32),
        pltpu.VMEM((L, D), jnp.float32),
        pltpu.VMEM((5, D), jnp.float32),
        [pltpu.SemaphoreType.DMA] * NSUB,
        [pltpu.SemaphoreType.DMA] * NSUB,
        [pltpu.SemaphoreType.DMA] * NSUB,
        [pltpu.SemaphoreType.DMA] * NSUB,
    ],
    compiler_params=pltpu.CompilerParams(needs_layout_passes=False),
)
def _embed_kernel(tok_hbm, posf_hbm, tid_hbm, pid_hbm, out_hbm,
                  pos_v, tidc, pidc, buf, bufT, bufE, sit, sip, sg, so):
    wid = lax.axis_index("s") * NC + lax.axis_index("c")
    bb0 = wid * BB_PER_W
    iota = lax.iota(jnp.int32, L)

    pltpu.sync_copy(posf_hbm, pos_v)

    def start_idx(si, bbg):
        pltpu.async_copy(tid_hbm.at[bbg, pl.ds(W0[si], L)],
                         tidc.at[pl.ds(si * L, L)], sit[si])
        pltpu.async_copy(pid_hbm.at[bbg, pl.ds(W0[si], L)],
                         pidc.at[pl.ds(si * L, L)], sip[si])

    def wait_idx_t(si):
        pltpu.make_async_copy(tid_hbm.at[0, pl.ds(0, L)],
                              tidc.at[pl.ds(si * L, L)], sit[si]).wait()

    def wait_idx_p(si):
        pltpu.make_async_copy(pid_hbm.at[0, pl.ds(0, L)],
                              pidc.at[pl.ds(si * L, L)], sip[si]).wait()

    def start_gather(si):
        if si < 4:
            pltpu.async_copy(tok_hbm.at[tidc.at[pl.ds(si * L, L)]],
                             buf.at[si], sg[si])
        else:
            pltpu.async_copy(tok_hbm.at[tidc.at[pl.ds(si * L, L)]],
                             bufT, sg[si])

    def wait_gather(si):
        if si < 4:
            pltpu.make_async_copy(tok_hbm.at[pl.ds(0, L)], buf.at[si],
                                  sg[si]).wait()
        else:
            pltpu.make_async_copy(tok_hbm.at[pl.ds(0, L)],
                                  bufT, sg[si]).wait()

    def start_out(si, bbg):
        if si < 4:
            pltpu.async_copy(buf.at[si], out_hbm.at[bbg, pl.ds(W0[si], L)],
                             so[si])
        else:
            pltpu.async_copy(bufT.at[pl.ds(0, 8)],
                             out_hbm.at[bbg, pl.ds(W0[4], 8)], so[si])
            pltpu.async_copy(bufE, out_hbm.at[bbg, pl.ds(W0[4] + 8, 5)],
                             so[si])

    def wait_out(si):
        if si < 4:
            pltpu.make_async_copy(buf.at[si],
                                  out_hbm.at[0, pl.ds(W0[si], L)],
                                  so[si]).wait()
        else:
            pltpu.make_async_copy(bufT.at[pl.ds(0, 8)],
                                  out_hbm.at[0, pl.ds(W0[4], 8)],
                                  so[si]).wait()
            pltpu.make_async_copy(bufE, out_hbm.at[0, pl.ds(W0[4] + 8, 5)],
                                  so[si]).wait()

    def do_add(si):
        if si < 4:
            def row(r, cc):
                rv = jnp.full((L,), si * L + r, jnp.int32)
                pidv = plsc.load_gather(pidc, [rv])
                bvec = pidv * D + iota
                for j in range(D // L):
                    v = plsc.load_gather(pos_v, [bvec + j * L])
                    plsc.addupdate(buf.at[si, r, pl.ds(j * L, L)], v)
                return cc

            lax.fori_loop(0, L, row, 0)
        else:
            def rowA(r, cc):
                rv = jnp.full((L,), si * L + r, jnp.int32)
                pidv = plsc.load_gather(pidc, [rv])
                bvec = pidv * D + iota
                for j in range(D // L):
                    v = plsc.load_gather(pos_v, [bvec + j * L])
                    plsc.addupdate(bufT.at[r, pl.ds(j * L, L)], v)
                return cc

            lax.fori_loop(0, 8, rowA, 0)

            def rowB(r, cc):
                rv = jnp.full((L,), si * L + r, jnp.int32)
                pidv = plsc.load_gather(pidc, [rv])
                bvec = pidv * D + iota
                for j in range(D // L):
                    v = plsc.load_gather(pos_v, [bvec + j * L])
                    sl = pl.ds(j * L, L)
                    bufE[r - 8, sl] = bufT[r, sl] + v
                return cc

            lax.fori_loop(8, TK, rowB, 0)

    for si in range(NSUB):
        start_idx(si, bb0)
    for si in range(3):
        wait_idx_t(si)
        start_gather(si)

    def outer(bb, carry):
        bbg = bb0 + bb
        for si in range(NSUB):
            c = bb * NSUB + si
            si3 = (si + 3) % NSUB

            @pl.when(jnp.logical_and(c + 3 < NCHT, c >= 2))
            def _():
                wait_out(si3)

            @pl.when(c + 3 < NCHT)
            def _():
                wait_idx_t(si3)
                start_gather(si3)

            wait_gather(si)
            wait_idx_p(si)
            do_add(si)
            start_out(si, bbg)

            @pl.when(c + NSUB < NCHT)
            def _():
                start_idx(si, bbg + 1)
        return carry

    lax.fori_loop(0, BB_PER_W, outer, 0)
    for si in range(NSUB):
        wait_out(si)


def kernel(input_ids, position_ids, token_embedding, position_embedding):
    tid = jnp.pad(input_ids.astype(jnp.int32), ((0, 0), (0, NPAD - NPOS)))
    pid = jnp.pad(position_ids.astype(jnp.int32), ((0, 0), (0, NPAD - NPOS)))
    return _embed_kernel(token_embedding, position_embedding.reshape(-1),
                         tid, pid)


# speedup vs baseline: 1.4402x; 1.0002x over previous
import functools

import jax
import jax.numpy as jnp
from jax import lax
from jax.experimental import pallas as pl
from jax.experimental.pallas import tpu as pltpu
from jax.experimental.pallas import tpu_sc as plsc

VOCAB = 49408
NPOS = 77
NPAD = 80
D = 768
B = 4096
NC, NS, L = 2, 16, 16
NWORK = NC * NS
BB_PER_W = B // NWORK
NSUB = 5
W0 = (0, 16, 32, 48, 64)
TK = 13
NCHT = BB_PER_W * NSUB

_mesh = plsc.VectorSubcoreMesh(core_axis_name="c", subcore_axis_name="s")


@functools.partial(
    pl.kernel,
    mesh=_mesh,
    out_type=jax.ShapeDtypeStruct((B, NPOS, D), jnp.float32),
    scratch_types=[
        pltpu.VMEM((NPOS * D,), jnp.float32),
        pltpu.VMEM((NSUB * L,), jnp.int32),
        pltpu.VMEM((NSUB * L,), jnp.int32),
        pltpu.VMEM((4, L, D), jnp.float32),
        pltpu.VMEM((L, D), jnp.float32),
        pltpu.VMEM((5, D), jnp.float32),
        [pltpu.SemaphoreType.DMA] * NSUB,
        [pltpu.SemaphoreType.DMA] * NSUB,
        [pltpu.SemaphoreType.DMA] * NSUB,
        [pltpu.SemaphoreType.DMA] * NSUB,
    ],
    compiler_params=pltpu.CompilerParams(
        needs_layout_passes=False, skip_device_barrier=True),
)
def _embed_kernel(tok_hbm, posf_hbm, tid_hbm, pid_hbm, out_hbm,
                  pos_v, tidc, pidc, buf, bufT, bufE, sit, sip, sg, so):
    wid = lax.axis_index("s") * NC + lax.axis_index("c")
    bb0 = wid * BB_PER_W
    iota = lax.iota(jnp.int32, L)

    pltpu.sync_copy(posf_hbm, pos_v)

    def start_idx(si, bbg):
        pltpu.async_copy(tid_hbm.at[bbg, pl.ds(W0[si], L)],
                         tidc.at[pl.ds(si * L, L)], sit[si])
        pltpu.async_copy(pid_hbm.at[bbg, pl.ds(W0[si], L)],
                         pidc.at[pl.ds(si * L, L)], sip[si])

    def wait_idx_t(si):
        pltpu.make_async_copy(tid_hbm.at[0, pl.ds(0, L)],
                              tidc.at[pl.ds(si * L, L)], sit[si]).wait()

    def wait_idx_p(si):
        pltpu.make_async_copy(pid_hbm.at[0, pl.ds(0, L)],
                              pidc.at[pl.ds(si * L, L)], sip[si]).wait()

    def start_gather(si):
        if si < 4:
            pltpu.async_copy(tok_hbm.at[tidc.at[pl.ds(si * L, L)]],
                             buf.at[si], sg[si])
        else:
            pltpu.async_copy(tok_hbm.at[tidc.at[pl.ds(si * L, L)]],
                             bufT, sg[si])

    def wait_gather(si):
        if si < 4:
            pltpu.make_async_copy(tok_hbm.at[pl.ds(0, L)], buf.at[si],
                                  sg[si]).wait()
        else:
            pltpu.make_async_copy(tok_hbm.at[pl.ds(0, L)],
                                  bufT, sg[si]).wait()

    def start_out(si, bbg):
        if si < 4:
            pltpu.async_copy(buf.at[si], out_hbm.at[bbg, pl.ds(W0[si], L)],
                             so[si])
        else:
            pltpu.async_copy(bufT.at[pl.ds(0, 8)],
                             out_hbm.at[bbg, pl.ds(W0[4], 8)], so[si])
            pltpu.async_copy(bufE, out_hbm.at[bbg, pl.ds(W0[4] + 8, 5)],
                             so[si])

    def wait_out(si):
        if si < 4:
            pltpu.make_async_copy(buf.at[si],
                                  out_hbm.at[0, pl.ds(W0[si], L)],
                                  so[si]).wait()
        else:
            pltpu.make_async_copy(bufT.at[pl.ds(0, 8)],
                                  out_hbm.at[0, pl.ds(W0[4], 8)],
                                  so[si]).wait()
            pltpu.make_async_copy(bufE, out_hbm.at[0, pl.ds(W0[4] + 8, 5)],
                                  so[si]).wait()

    def do_add(si):
        if si < 4:
            def row(r, cc):
                rv = jnp.full((L,), si * L + r, jnp.int32)
                pidv = plsc.load_gather(pidc, [rv])
                bvec = pidv * D + iota
                for j in range(D // L):
                    v = plsc.load_gather(pos_v, [bvec + j * L])
                    plsc.addupdate(buf.at[si, r, pl.ds(j * L, L)], v)
                return cc

            lax.fori_loop(0, L, row, 0)
        else:
            def rowA(r, cc):
                rv = jnp.full((L,), si * L + r, jnp.int32)
                pidv = plsc.load_gather(pidc, [rv])
                bvec = pidv * D + iota
                for j in range(D // L):
                    v = plsc.load_gather(pos_v, [bvec + j * L])
                    plsc.addupdate(bufT.at[r, pl.ds(j * L, L)], v)
                return cc

            lax.fori_loop(0, 8, rowA, 0)

            def rowB(r, cc):
                rv = jnp.full((L,), si * L + r, jnp.int32)
                pidv = plsc.load_gather(pidc, [rv])
                bvec = pidv * D + iota
                for j in range(D // L):
                    v = plsc.load_gather(pos_v, [bvec + j * L])
                    sl = pl.ds(j * L, L)
                    bufE[r - 8, sl] = bufT[r, sl] + v
                return cc

            lax.fori_loop(8, TK, rowB, 0)

    for si in range(NSUB):
        start_idx(si, bb0)
    for si in range(3):
        wait_idx_t(si)
        start_gather(si)

    def outer(bb, carry):
        bbg = bb0 + bb
        for si in range(NSUB):
            c = bb * NSUB + si
            si3 = (si + 3) % NSUB

            @pl.when(jnp.logical_and(c + 3 < NCHT, c >= 2))
            def _():
                wait_out(si3)

            @pl.when(c + 3 < NCHT)
            def _():
                wait_idx_t(si3)
                start_gather(si3)

            wait_gather(si)
            wait_idx_p(si)
            do_add(si)
            start_out(si, bbg)

            @pl.when(c + NSUB < NCHT)
            def _():
                start_idx(si, bbg + 1)
        return carry

    lax.fori_loop(0, BB_PER_W, outer, 0)
    for si in range(NSUB):
        wait_out(si)


def kernel(input_ids, position_ids, token_embedding, position_embedding):
    tid = jnp.pad(input_ids.astype(jnp.int32), ((0, 0), (0, NPAD - NPOS)))
    pid = jnp.pad(position_ids.astype(jnp.int32), ((0, 0), (0, NPAD - NPOS)))
    return _embed_kernel(token_embedding, position_embedding.reshape(-1),
                         tid, pid)
